# Initial kernel scaffold; baseline (speedup 1.0000x reference)
import jax
import jax.numpy as jnp
from jax.experimental import pallas as pl


def kernel(edge_index, edge_values, x, W, b):
    raise NotImplementedError("write your pallas kernel here")


# speedup vs baseline: 5.6372x; 5.6372x over previous
import functools

import jax
import jax.numpy as jnp
from jax import lax
from jax.experimental import pallas as pl
from jax.experimental.pallas import tpu as pltpu
from jax.experimental.pallas import tpu_sc as plsc

N = 10000
E = 320000
D = 128

NC = 2
NS = 16
L = 16
NW = NC * NS

CHUNK = 128
CHUNKS_TOTAL = E // CHUNK
RBLK = 80
NRBLK = N // RBLK


def _matmul_body(x_ref, w_ref, b_ref, o_ref):
    o_ref[...] = lax.dot_general(
        x_ref[...], w_ref[...], (((1,), (1,)), ((), ())),
        preferred_element_type=jnp.float32,
    ) + b_ref[...]


def _matmul(x, W, b2d):
    grid = 10
    bm = N // grid
    return pl.pallas_call(
        _matmul_body,
        grid=(grid,),
        in_specs=[
            pl.BlockSpec((bm, D), lambda i: (i, 0)),
            pl.BlockSpec((D, D), lambda i: (0, 0)),
            pl.BlockSpec((1, D), lambda i: (0, 0)),
        ],
        out_specs=pl.BlockSpec((bm, D), lambda i: (i, 0)),
        out_shape=jax.ShapeDtypeStruct((N, D), jnp.float32),
    )(x, W, b2d)


def _edge_body(edge_hbm, val_hbm, xw_hbm, out_hbm,
               acc, idx_v, dst_v, val_v, rows_v, zero_v, sem):
    cid = lax.axis_index("c")
    sid = lax.axis_index("s")
    wid = sid * NC + cid

    def _zrow(i, carry):
        for j in range(D // L):
            zero_v[i, pl.ds(j * L, L)] = jnp.zeros((L,), jnp.float32)
        return carry

    lax.fori_loop(0, RBLK, _zrow, 0)

    base_rblk = NRBLK // NS
    nrblk = jnp.where(sid < NRBLK % NS, base_rblk + 1, base_rblk)

    def _zblk(k, carry):
        pltpu.sync_copy(zero_v, acc.at[pl.ds((sid + k * NS) * RBLK, RBLK)])
        return carry

    lax.fori_loop(0, nrblk, _zblk, 0)
    plsc.subcore_barrier()

    base_chunks = CHUNKS_TOTAL // NW
    nchunks = jnp.where(wid < CHUNKS_TOTAL % NW, base_chunks + 1, base_chunks)

    def _chunk(j, carry):
        base = (wid + j * NW) * CHUNK
        pltpu.sync_copy(edge_hbm.at[1, pl.ds(base, CHUNK)], idx_v)
        pltpu.sync_copy(edge_hbm.at[0, pl.ds(base, CHUNK)], dst_v)
        pltpu.sync_copy(val_hbm.at[pl.ds(base, CHUNK)], val_v)
        pltpu.async_copy(xw_hbm.at[idx_v], rows_v, sem).wait()

        def _sgrp(g, c2):
            vals = val_v[pl.ds(g * L, L)]
            for ii in range(L):
                v = vals[ii]
                i = g * L + ii
                for jj in range(D // L):
                    rows_v[i, pl.ds(jj * L, L)] = rows_v[i, pl.ds(jj * L, L)] * v
            return c2

        lax.fori_loop(0, CHUNK // L, _sgrp, 0)
        pltpu.sync_copy(rows_v, acc.at[dst_v], add=True)
        return carry

    lax.fori_loop(0, nchunks, _chunk, 0)
    plsc.subcore_barrier()

    def _cblk(k, carry):
        row0 = (sid + k * NS) * RBLK
        pltpu.sync_copy(acc.at[pl.ds(row0, RBLK)],
                        out_hbm.at[cid, pl.ds(row0, RBLK)])
        return carry

    lax.fori_loop(0, nrblk, _cblk, 0)


_edge_pass = functools.partial(
    pl.kernel,
    out_type=jax.ShapeDtypeStruct((NC, N, D), jnp.float32),
    mesh=plsc.VectorSubcoreMesh(
        core_axis_name="c", subcore_axis_name="s",
        num_cores=NC, num_subcores=NS,
    ),
    scratch_types=[
        pltpu.VMEM_SHARED((N, D), jnp.float32),
        pltpu.VMEM((CHUNK,), jnp.int32),
        pltpu.VMEM((CHUNK,), jnp.int32),
        pltpu.VMEM((CHUNK,), jnp.float32),
        pltpu.VMEM((CHUNK, D), jnp.float32),
        pltpu.VMEM((RBLK, D), jnp.float32),
        pltpu.SemaphoreType.DMA,
    ],
)(_edge_body)


def _combine_body(p_ref, o_ref):
    o_ref[...] = jnp.maximum(p_ref[0] + p_ref[1], 0.0)


def _combine(partial):
    grid = 10
    bm = N // grid
    return pl.pallas_call(
        _combine_body,
        grid=(grid,),
        in_specs=[pl.BlockSpec((NC, bm, D), lambda i: (0, i, 0))],
        out_specs=pl.BlockSpec((bm, D), lambda i: (i, 0)),
        out_shape=jax.ShapeDtypeStruct((N, D), jnp.float32),
    )(partial)


def kernel(edge_index, edge_values, x, W, b):
    xw = _matmul(x, W, b.reshape(1, D))
    partial = _edge_pass(edge_index, edge_values, xw)
    return _combine(partial)
